# baseline (speedup 1.0000x reference)
import functools

import jax
import jax.numpy as jnp
from jax import lax
from jax.experimental import pallas as pl
from jax.experimental.pallas import tpu as pltpu
from jax.experimental.pallas import tpu_sc as plsc

_NC = 2
_NS = 16
_NW = _NC * _NS
_CL = 16
_LANES = 16


def kernel(x, pe):
    B, L, E = x.shape
    l_per_w = L // _NW
    n_chunks = l_per_w // _CL
    vecs = (_CL * E) // _LANES
    xf = x.reshape(B * L * E)
    pef = pe.reshape(pe.shape[0] * E)

    mesh = plsc.VectorSubcoreMesh(core_axis_name="c", subcore_axis_name="s")

    @functools.partial(
        pl.kernel,
        mesh=mesh,
        out_type=jax.ShapeDtypeStruct((B * L * E,), jnp.float32),
        scratch_types=[
            pltpu.VMEM((_CL * E,), jnp.float32),
            pltpu.VMEM((B, _CL * E), jnp.float32),
            pltpu.SemaphoreType.DMA,
            pltpu.SemaphoreType.DMA,
        ],
    )
    def sc_k(x_hbm, pe_hbm, o_hbm, pe_v, buf_v, sem_in, sem_out):
        wid = lax.axis_index("s") * _NC + lax.axis_index("c")
        for c in range(n_chunks):
            l0 = wid * l_per_w + c * _CL
            loads = [
                pltpu.async_copy(
                    pe_hbm.at[pl.ds((l0 + 1) * E, _CL * E)], pe_v, sem_in
                )
            ]
            for b in range(B):
                loads.append(
                    pltpu.async_copy(
                        x_hbm.at[pl.ds((b * L + l0) * E, _CL * E)],
                        buf_v.at[b],
                        sem_in,
                    )
                )
            for cp in loads:
                cp.wait()

            def body(i, _):
                for u in range(4):
                    off = (i * 4 + u) * _LANES
                    pv = pe_v[pl.ds(off, _LANES)]
                    for b in range(B):
                        buf_v[b, pl.ds(off, _LANES)] = (
                            buf_v[b, pl.ds(off, _LANES)] + pv
                        )
                return 0

            lax.fori_loop(0, vecs // 4, body, 0)

            stores = [
                pltpu.async_copy(
                    buf_v.at[b],
                    o_hbm.at[pl.ds((b * L + l0) * E, _CL * E)],
                    sem_out,
                )
                for b in range(B)
            ]
            for cp in stores:
                cp.wait()

    return sc_k(xf, pef).reshape(B, L, E)


# speedup vs baseline: 1.3942x; 1.3942x over previous
import functools

import jax
import jax.numpy as jnp
from jax import lax
from jax.experimental import pallas as pl
from jax.experimental.pallas import tpu as pltpu
from jax.experimental.pallas import tpu_sc as plsc

_NC = 2
_NS = 16
_NW = _NC * _NS
_CL = 16
_LANES = 16


def kernel(x, pe):
    B, L, E = x.shape
    l_per_w = L // _NW
    n_chunks = l_per_w // _CL
    vecs = (_CL * E) // _LANES
    xf = x.reshape(B * L * E)
    pef = pe.reshape(pe.shape[0] * E)

    mesh = plsc.VectorSubcoreMesh(core_axis_name="c", subcore_axis_name="s")

    @functools.partial(
        pl.kernel,
        mesh=mesh,
        out_type=jax.ShapeDtypeStruct((B * L * E,), jnp.float32),
        scratch_types=[
            pltpu.VMEM((_CL * E,), jnp.float32),
            pltpu.VMEM((B, _CL * E), jnp.float32),
            pltpu.SemaphoreType.DMA,
            pltpu.SemaphoreType.DMA,
        ],
    )
    def sc_k(x_hbm, pe_hbm, o_hbm, pe_v, buf_v, sem_in, sem_out):
        wid = lax.axis_index("s") * _NC + lax.axis_index("c")
        for c in range(n_chunks):
            l0 = wid * l_per_w + c * _CL
            loads = [
                pltpu.async_copy(
                    pe_hbm.at[pl.ds((l0 + 1) * E, _CL * E)], pe_v, sem_in
                )
            ]
            for b in range(B):
                loads.append(
                    pltpu.async_copy(
                        x_hbm.at[pl.ds((b * L + l0) * E, _CL * E)],
                        buf_v.at[b],
                        sem_in,
                    )
                )
            for cp in loads:
                cp.wait()

            @plsc.parallel_loop(0, vecs, 1, unroll=8)
            def body(j):
                off = j * _LANES
                pv = pe_v[pl.ds(off, _LANES)]
                for b in range(B):
                    buf_v[b, pl.ds(off, _LANES)] = (
                        buf_v[b, pl.ds(off, _LANES)] + pv
                    )

            stores = [
                pltpu.async_copy(
                    buf_v.at[b],
                    o_hbm.at[pl.ds((b * L + l0) * E, _CL * E)],
                    sem_out,
                )
                for b in range(B)
            ]
            for cp in stores:
                cp.wait()

    return sc_k(xf, pef).reshape(B, L, E)
